# Initial kernel scaffold; baseline (speedup 1.0000x reference)
import jax
import jax.numpy as jnp
from jax.experimental import pallas as pl


def kernel(support_features, support_labels, query_features):
    raise NotImplementedError("write your pallas kernel here")


# speedup vs baseline: 1.4340x; 1.4340x over previous
import functools

import jax
import jax.numpy as jnp
from jax import lax
from jax.experimental import pallas as pl
from jax.experimental.pallas import tpu as pltpu
from jax.experimental.pallas import tpu_sc as plsc

NUM_CLASSES = 64
N_SUPPORT = 8192
N_QUERY = 16384
D_FEAT = 512

NC = 2
NS = 16
NW = NC * NS
ROWS_PER_TILE = N_SUPPORT // NW
SC_BATCH = 128
CNT_W = 128


def _sc_segment_sums(support_features, support_labels):
    zeros_sum = jnp.zeros((NUM_CLASSES * D_FEAT,), jnp.float32)
    zeros_cnt = jnp.zeros((NUM_CLASSES * CNT_W,), jnp.float32)

    mesh = plsc.VectorSubcoreMesh(core_axis_name="c", subcore_axis_name="s",
                                  num_cores=NC, num_subcores=NS)

    @functools.partial(
        pl.kernel,
        out_type=(
            jax.ShapeDtypeStruct((NW, NUM_CLASSES * D_FEAT), jnp.float32),
            jax.ShapeDtypeStruct((NW, NUM_CLASSES * CNT_W), jnp.float32),
        ),
        mesh=mesh,
        scratch_types=[
            pltpu.VMEM((SC_BATCH, D_FEAT), jnp.float32),
            pltpu.VMEM((SC_BATCH,), jnp.int32),
            pltpu.VMEM((NUM_CLASSES * D_FEAT,), jnp.float32),
            pltpu.VMEM((NUM_CLASSES * CNT_W,), jnp.float32),
        ],
        compiler_params=pltpu.CompilerParams(use_tc_tiling_on_sc=False,
                                             needs_layout_passes=False),
    )
    def seg_kernel(feat_hbm, lab_hbm, zsum_hbm, zcnt_hbm,
                   out_sums, out_cnts,
                   rowbuf, labbuf, acc_sum, acc_cnt):
        c = lax.axis_index("c")
        s = lax.axis_index("s")

        pltpu.sync_copy(zsum_hbm, acc_sum)
        pltpu.sync_copy(zcnt_hbm, acc_cnt)

        col0 = lax.iota(jnp.int32, 16)
        ones_v = jnp.full((16,), 1.0, jnp.float32)

        wid = s * NC + c
        base = wid * ROWS_PER_TILE
        for b in range(ROWS_PER_TILE // SC_BATCH):
            off = base + b * SC_BATCH
            pltpu.sync_copy(lab_hbm.at[pl.ds(off, SC_BATCH)], labbuf)
            pltpu.sync_copy(feat_hbm.at[pl.ds(off, SC_BATCH)], rowbuf)

            def body(i, _):
                labs16 = labbuf[pl.ds(i * 16, 16)]
                plsc.addupdate_scatter(
                    acc_cnt, [labs16 * CNT_W + col0], ones_v)
                for j in range(16):
                    labj = jnp.take(labs16, jnp.full((16,), j, jnp.int32))
                    rowbase = labj * D_FEAT + col0
                    for kk in range(D_FEAT // 16):
                        chunk = rowbuf[i * 16 + j, pl.ds(kk * 16, 16)]
                        plsc.addupdate_scatter(
                            acc_sum, [rowbase + (kk * 16)], chunk)
                return 0
            lax.fori_loop(0, SC_BATCH // 16, body, 0, unroll=False)

        pltpu.sync_copy(acc_sum, out_sums.at[wid])
        pltpu.sync_copy(acc_cnt, out_cnts.at[wid])

    psums, pcnts = seg_kernel(support_features, support_labels,
                              zeros_sum, zeros_cnt)
    return (psums.reshape(NW, NUM_CLASSES, D_FEAT),
            pcnts.reshape(NW, NUM_CLASSES, CNT_W))


BQ = 1024


def _tc_body(psums_ref, pcnts_ref, q_ref, out_ref, proto_ref, p2_ref):
    @pl.when(pl.program_id(0) == 0)
    def _():
        sums = jnp.sum(psums_ref[...], axis=0)
        cnt_t = jnp.sum(pcnts_ref[...], axis=0)
        cnts = jnp.sum(cnt_t, axis=1, keepdims=True)
        proto = sums / cnts
        proto_ref[...] = proto
        p2_ref[...] = jnp.sum(proto * proto, axis=1)[None, :]

    q = q_ref[...]
    q2 = jnp.sum(q * q, axis=1, keepdims=True)
    qp = lax.dot_general(q, proto_ref[...], (((1,), (1,)), ((), ())),
                         preferred_element_type=jnp.float32)
    d2 = q2 + p2_ref[...] - 2.0 * qp
    out_ref[...] = -jnp.sqrt(jnp.maximum(d2, 0.0))


def kernel(support_features, support_labels, query_features):
    psums, pcnts = _sc_segment_sums(support_features, support_labels)
    grid = (N_QUERY // BQ,)
    return pl.pallas_call(
        _tc_body,
        grid=grid,
        in_specs=[
            pl.BlockSpec((NW, NUM_CLASSES, D_FEAT), lambda i: (0, 0, 0)),
            pl.BlockSpec((NW, NUM_CLASSES, CNT_W), lambda i: (0, 0, 0)),
            pl.BlockSpec((BQ, D_FEAT), lambda i: (i, 0)),
        ],
        out_specs=pl.BlockSpec((BQ, NUM_CLASSES), lambda i: (i, 0)),
        out_shape=jax.ShapeDtypeStruct((N_QUERY, NUM_CLASSES), jnp.float32),
        scratch_shapes=[
            pltpu.VMEM((NUM_CLASSES, D_FEAT), jnp.float32),
            pltpu.VMEM((1, NUM_CLASSES), jnp.float32),
        ],
        compiler_params=pltpu.CompilerParams(
            dimension_semantics=("arbitrary",),
        ),
    )(psums, pcnts, query_features)
